# baseline (speedup 1.0000x reference)
import functools

import jax
import jax.numpy as jnp
from jax import lax
from jax.experimental import pallas as pl
from jax.experimental.pallas import tpu as pltpu
from jax.experimental.pallas import tpu_sc as plsc

B = 4096
L = 200
D = 64
LP = 224
H = LP // 2
CH = H // 16
NC = 2
NS = 16
NW = NC * NS
RPW = B // NW
NV = D // 16


def _sc_call(inputs3, emb_table, wtab):
    mesh = plsc.VectorSubcoreMesh(core_axis_name="c", subcore_axis_name="s")

    @functools.partial(
        pl.kernel,
        out_type=jax.ShapeDtypeStruct((B, D), jnp.float32),
        mesh=mesh,
        scratch_types=[
            pltpu.VMEM((2, H), jnp.int32),
            pltpu.VMEM((H, D), jnp.float32),
            pltpu.VMEM((H, D), jnp.float32),
            pltpu.VMEM((H,), jnp.float32),
            pltpu.VMEM((H,), jnp.float32),
            pltpu.VMEM((D,), jnp.float32),
            pltpu.SemaphoreType.DMA,
        ],
        compiler_params=pltpu.CompilerParams(use_tc_tiling_on_sc=False),
    )
    def k(inputs_hbm, emb_hbm, w_hbm, out_hbm,
          idx_v, rows0_v, rows1_v, w0_v, w1_v, acc_v, sem):
        wid = lax.axis_index("s") * NC + lax.axis_index("c")
        base = wid * RPW
        rows_v = (rows0_v, rows1_v)
        w_v = (w0_v, w1_v)

        def row_body(b, carry):
            r = base + b
            pltpu.sync_copy(inputs_hbm.at[r], idx_v)
            cps = []
            for h in range(2):
                cps.append(pltpu.async_copy(
                    emb_hbm.at[idx_v.at[h]], rows_v[h], sem))
                cps.append(pltpu.async_copy(
                    w_hbm.at[idx_v.at[h]], w_v[h], sem))
            for cp in cps:
                cp.wait()

            acc = tuple(jnp.zeros((16,), jnp.float32) for _ in range(NV))
            for h in range(2):
                def c_body(c, acc, h=h):
                    l0 = c * 16
                    w16 = w_v[h][pl.ds(l0, 16)]
                    acc = list(acc)
                    for i in range(16):
                        wi = w16[i]
                        for kv in range(NV):
                            acc[kv] = acc[kv] + wi * rows_v[h][
                                l0 + i, pl.ds(kv * 16, 16)]
                    return tuple(acc)

                acc = lax.fori_loop(0, CH, c_body, acc)

            for kv in range(NV):
                acc_v[pl.ds(kv * 16, 16)] = acc[kv]
            pltpu.sync_copy(acc_v, out_hbm.at[r])
            return carry

        lax.fori_loop(0, RPW, row_body, 0)

    return k(inputs3, emb_table, wtab)


def kernel(inputs, emb_table, weight_table):
    wtab = weight_table.at[0, 0].set(0.0).reshape(-1)
    inputs3 = jnp.pad(inputs, ((0, 0), (0, LP - L))).reshape(B, 2, H)
    return _sc_call(inputs3, emb_table, wtab)
